# baseline (speedup 1.0000x reference)
import functools

import jax
import jax.numpy as jnp
from jax.experimental import pallas as pl

_TOKEN_TILE = 256


def _rpq_kernel(x_ref, w_ref, cb_ref, out_ref):
    x = x_ref[...]
    w = w_ref[...]
    z = jax.lax.dot_general(
        x.astype(jnp.bfloat16), w.astype(jnp.bfloat16),
        (((1,), (1,)), ((), ())),
        preferred_element_type=jnp.float32,
    )
    mu = jnp.mean(z, axis=-1, keepdims=True)
    var = jnp.mean((z - mu) * (z - mu), axis=-1, keepdims=True)
    z = (z - mu) / jnp.sqrt(var + 1e-5)

    cb = cb_ref[...]
    s = jax.lax.dot_general(
        z.astype(jnp.bfloat16), cb.astype(jnp.bfloat16),
        (((1,), (1,)), ((), ())),
        preferred_element_type=jnp.float32,
    )
    t = z.shape[0]
    c2b = jax.lax.dot_general(
        jnp.ones((t, cb.shape[1]), jnp.float32), cb * cb,
        (((1,), (1,)), ((), ())),
        preferred_element_type=jnp.float32,
        precision=jax.lax.Precision.HIGHEST,
    )
    z2 = jnp.sum(z * z, axis=-1, keepdims=True)
    d2 = jnp.maximum(z2 - 2.0 * s + c2b, 0.0)

    m = jnp.min(d2, axis=-1, keepdims=True)
    k = d2.shape[1]
    iota = jax.lax.broadcasted_iota(jnp.int32, d2.shape, 1)
    idx = jnp.min(jnp.where(d2 == m, iota, k), axis=-1)
    out_ref[...] = idx


@functools.partial(jax.jit, static_argnames=())
def kernel(x, W, codebook):
    B, L, D = x.shape
    K, CD = codebook.shape
    n = B * L
    xf = x.reshape(n, D)
    tile = _TOKEN_TILE
    grid = (n // tile,)
    codes = pl.pallas_call(
        _rpq_kernel,
        grid=grid,
        in_specs=[
            pl.BlockSpec((tile, D), lambda i: (i, 0)),
            pl.BlockSpec((CD, D), lambda i: (0, 0)),
            pl.BlockSpec((K, CD), lambda i: (0, 0)),
        ],
        out_specs=pl.BlockSpec((tile,), lambda i: (i,)),
        out_shape=jax.ShapeDtypeStruct((n,), jnp.int32),
    )(xf, W, codebook)
    return codes.reshape(B, L)


# speedup vs baseline: 3.2668x; 3.2668x over previous
import functools

import jax
import jax.numpy as jnp
from jax.experimental import pallas as pl

_TOKEN_TILE = 256


def _rpq_kernel(x_ref, w_ref, cb_ref, out_ref):
    x = x_ref[...]
    w = w_ref[...]
    z = jax.lax.dot_general(
        x.astype(jnp.bfloat16), w.astype(jnp.bfloat16),
        (((1,), (1,)), ((), ())),
        preferred_element_type=jnp.float32,
    )
    mu = jnp.mean(z, axis=-1, keepdims=True)
    var = jnp.mean((z - mu) * (z - mu), axis=-1, keepdims=True)
    z = (z - mu) / jnp.sqrt(var + 1e-5)

    cb = cb_ref[...]
    c2 = jnp.sum(cb * cb, axis=-1, keepdims=True)
    t = z.shape[0]
    z_aug = jnp.concatenate(
        [(-2.0 * z).astype(jnp.bfloat16), jnp.ones((t, 1), jnp.bfloat16)], axis=1)
    cb_aug = jnp.concatenate([cb.astype(jnp.bfloat16), c2.astype(jnp.bfloat16)], axis=1)
    d = jax.lax.dot_general(
        z_aug, cb_aug, (((1,), (1,)), ((), ())),
        preferred_element_type=jnp.float32,
    )
    out_ref[...] = jnp.argmin(d, axis=-1).astype(jnp.int32)


@functools.partial(jax.jit, static_argnames=())
def kernel(x, W, codebook):
    B, L, D = x.shape
    K, CD = codebook.shape
    n = B * L
    xf = x.reshape(n, D)
    tile = _TOKEN_TILE
    grid = (n // tile,)
    codes = pl.pallas_call(
        _rpq_kernel,
        grid=grid,
        in_specs=[
            pl.BlockSpec((tile, D), lambda i: (i, 0)),
            pl.BlockSpec((CD, D), lambda i: (0, 0)),
            pl.BlockSpec((K, CD), lambda i: (0, 0)),
        ],
        out_specs=pl.BlockSpec((tile,), lambda i: (i,)),
        out_shape=jax.ShapeDtypeStruct((n,), jnp.int32),
    )(xf, W, codebook)
    return codes.reshape(B, L)
